# baseline (speedup 1.0000x reference)
import functools

import jax
import jax.numpy as jnp
from jax import lax
from jax.experimental import pallas as pl
from jax.experimental.pallas import tpu as pltpu
from jax.experimental.pallas import tpu_sc as plsc

N_NODES = 10000
DEG = 32
D = 128

NC = 2
NS = 16
NW = NC * NS
LANES = 16
NCHUNK = D // LANES

RB = 8
BLK_ROWS = RB * DEG
NBLK = N_NODES // RB
STEPS = -(-NBLK // NW)

_sc_mesh = plsc.VectorSubcoreMesh(core_axis_name="c", subcore_axis_name="s")


def _sc_mean_body(neigh_hbm, out_hbm, buf, outb, isem0, isem1, osem0, osem1):
    wid = lax.axis_index("s") * NC + lax.axis_index("c")
    isems = (isem0, isem1)
    osems = (osem0, osem1)

    def blk_of(step):
        return jnp.minimum(wid + step * NW, NBLK - 1)

    def in_copy(step, b):
        return pltpu.make_async_copy(
            neigh_hbm.at[pl.ds(blk_of(step) * BLK_ROWS, BLK_ROWS), :],
            buf.at[b], isems[b])

    def out_copy(step, b):
        return pltpu.make_async_copy(
            outb.at[b], out_hbm.at[pl.ds(blk_of(step) * RB, RB), :], osems[b])

    in_copy(0, 0).start()

    def reduce_block(b):
        bufb = buf.at[b]
        outbb = outb.at[b]

        def row_body(r, carry):
            base = r * DEG
            accs = [bufb[base, pl.ds(LANES * j, LANES)] for j in range(NCHUNK)]
            for k in range(1, DEG):
                for j in range(NCHUNK):
                    accs[j] = accs[j] + bufb[base + k, pl.ds(LANES * j, LANES)]
            for j in range(NCHUNK):
                outbb[r, pl.ds(LANES * j, LANES)] = accs[j] * (1.0 / DEG)
            return carry

        lax.fori_loop(0, RB, row_body, 0)

    def outer(g, carry):
        for b in range(2):
            step = 2 * g + b
            in_copy(step + 1, 1 - b).start()
            in_copy(step, b).wait()
            pl.when(g > 0)(lambda: out_copy(step, b).wait())
            reduce_block(b)
            out_copy(step, b).start()
        return carry

    lax.fori_loop(0, STEPS // 2, outer, 0)

    in_copy(STEPS, 0).wait()
    out_copy(STEPS - 2, 0).wait()
    out_copy(STEPS - 1, 1).wait()


@jax.jit
def _sc_mean(neigh):
    return pl.kernel(
        _sc_mean_body,
        mesh=_sc_mesh,
        out_type=jax.ShapeDtypeStruct((N_NODES, D), jnp.float32),
        scratch_types=[
            pltpu.VMEM((2, BLK_ROWS, D), jnp.float32),
            pltpu.VMEM((2, RB, D), jnp.float32),
            pltpu.SemaphoreType.DMA,
            pltpu.SemaphoreType.DMA,
            pltpu.SemaphoreType.DMA,
            pltpu.SemaphoreType.DMA,
        ],
    )(neigh)


BN = 400


def _linear_body(x_ref, agg_ref, wx_ref, bx_ref, wn_ref, bn_ref, out_ref):
    h_x = lax.dot_general(
        x_ref[...], wx_ref[...], (((1,), (1,)), ((), ())),
        preferred_element_type=jnp.float32)
    h_n = lax.dot_general(
        agg_ref[...], wn_ref[...], (((1,), (1,)), ((), ())),
        preferred_element_type=jnp.float32)
    out_ref[:, :D] = jnp.maximum(h_x + bx_ref[...], 0.0)
    out_ref[:, D:] = jnp.maximum(h_n + bn_ref[...], 0.0)


@jax.jit
def _tc_linear(x, agg, W_x, b_x, W_n, b_n):
    return pl.pallas_call(
        _linear_body,
        grid=(N_NODES // BN,),
        in_specs=[
            pl.BlockSpec((BN, D), lambda i: (i, 0)),
            pl.BlockSpec((BN, D), lambda i: (i, 0)),
            pl.BlockSpec((D, D), lambda i: (0, 0)),
            pl.BlockSpec((1, D), lambda i: (0, 0)),
            pl.BlockSpec((D, D), lambda i: (0, 0)),
            pl.BlockSpec((1, D), lambda i: (0, 0)),
        ],
        out_specs=pl.BlockSpec((BN, 2 * D), lambda i: (i, 0)),
        out_shape=jax.ShapeDtypeStruct((N_NODES, 2 * D), jnp.float32),
    )(x, agg, W_x, b_x, W_n, b_n)


def kernel(x, neigh, W_x, b_x, W_n, b_n):
    agg = _sc_mean(neigh)
    return _tc_linear(x, agg, W_x.reshape(D, D), b_x.reshape(1, D),
                      W_n.reshape(D, D), b_n.reshape(1, D))


# speedup vs baseline: 1.5778x; 1.5778x over previous
import functools

import jax
import jax.numpy as jnp
from jax import lax
from jax.experimental import pallas as pl
from jax.experimental.pallas import tpu as pltpu
from jax.experimental.pallas import tpu_sc as plsc

N_NODES = 10000
DEG = 32
D = 128

S_SC = 4000
N_TC = N_NODES - S_SC

NC = 2
NS = 16
NW = NC * NS
LANES = 16
NCHUNK = D // LANES

RB = 8
BLK_ROWS = RB * DEG
NBLK = S_SC // RB
STEPS = 2 * (-(-NBLK // (2 * NW)))

_sc_mesh = plsc.VectorSubcoreMesh(core_axis_name="c", subcore_axis_name="s")


def _sc_mean_body(neigh_hbm, out_hbm, buf, outb, isem0, isem1, osem0, osem1):
    wid = lax.axis_index("s") * NC + lax.axis_index("c")
    isems = (isem0, isem1)
    osems = (osem0, osem1)

    def blk_of(step):
        return jnp.minimum(wid + step * NW, NBLK - 1)

    def in_copy(step, b):
        return pltpu.make_async_copy(
            neigh_hbm.at[pl.ds(blk_of(step) * BLK_ROWS, BLK_ROWS), :],
            buf.at[b], isems[b])

    def out_copy(step, b):
        return pltpu.make_async_copy(
            outb.at[b], out_hbm.at[pl.ds(blk_of(step) * RB, RB), :], osems[b])

    in_copy(0, 0).start()

    def reduce_block(b):
        bufb = buf.at[b]
        outbb = outb.at[b]

        def row_body(r, carry):
            base = r * DEG
            accs = [bufb[base, pl.ds(LANES * j, LANES)] for j in range(NCHUNK)]
            for k in range(1, DEG):
                for j in range(NCHUNK):
                    accs[j] = accs[j] + bufb[base + k, pl.ds(LANES * j, LANES)]
            for j in range(NCHUNK):
                outbb[r, pl.ds(LANES * j, LANES)] = accs[j] * (1.0 / DEG)
            return carry

        lax.fori_loop(0, RB, row_body, 0)

    def outer(g, carry):
        for b in range(2):
            step = 2 * g + b
            in_copy(step + 1, 1 - b).start()
            in_copy(step, b).wait()
            pl.when(g > 0)(lambda: out_copy(step, b).wait())
            reduce_block(b)
            out_copy(step, b).start()
        return carry

    lax.fori_loop(0, STEPS // 2, outer, 0)

    in_copy(STEPS, 0).wait()
    out_copy(STEPS - 2, 0).wait()
    out_copy(STEPS - 1, 1).wait()


_sc_mean = pl.kernel(
    _sc_mean_body,
    mesh=_sc_mesh,
    out_type=jax.ShapeDtypeStruct((S_SC, D), jnp.float32),
    scratch_types=[
        pltpu.VMEM((2, BLK_ROWS, D), jnp.float32),
        pltpu.VMEM((2, RB, D), jnp.float32),
        pltpu.SemaphoreType.DMA,
        pltpu.SemaphoreType.DMA,
        pltpu.SemaphoreType.DMA,
        pltpu.SemaphoreType.DMA,
    ],
)


BN = 400
OFF = S_SC // BN


def _fused_body(x_ref, neigh_ref, wx_ref, bx_ref, wn_ref, bn_ref, out_ref):
    nb = neigh_ref[...].reshape(BN, DEG, D)
    agg = jnp.sum(nb, axis=1) * (1.0 / DEG)
    h_x = lax.dot_general(
        x_ref[...], wx_ref[...], (((1,), (1,)), ((), ())),
        preferred_element_type=jnp.float32)
    h_n = lax.dot_general(
        agg, wn_ref[...], (((1,), (1,)), ((), ())),
        preferred_element_type=jnp.float32)
    out_ref[:, :D] = jnp.maximum(h_x + bx_ref[...], 0.0)
    out_ref[:, D:] = jnp.maximum(h_n + bn_ref[...], 0.0)


def _linear_body(prev_ref, x_ref, agg_ref, wx_ref, bx_ref, wn_ref, bn_ref,
                 out_ref):
    del prev_ref
    h_x = lax.dot_general(
        x_ref[...], wx_ref[...], (((1,), (1,)), ((), ())),
        preferred_element_type=jnp.float32)
    h_n = lax.dot_general(
        agg_ref[...], wn_ref[...], (((1,), (1,)), ((), ())),
        preferred_element_type=jnp.float32)
    out_ref[:, :D] = jnp.maximum(h_x + bx_ref[...], 0.0)
    out_ref[:, D:] = jnp.maximum(h_n + bn_ref[...], 0.0)


_wspecs = [
    pl.BlockSpec((D, D), lambda i: (0, 0)),
    pl.BlockSpec((1, D), lambda i: (0, 0)),
    pl.BlockSpec((D, D), lambda i: (0, 0)),
    pl.BlockSpec((1, D), lambda i: (0, 0)),
]


@jax.jit
def _run(x, neigh, W_x, b_x, W_n, b_n):
    agg_sc = _sc_mean(neigh)

    out1 = pl.pallas_call(
        _fused_body,
        grid=(N_TC // BN,),
        in_specs=[
            pl.BlockSpec((BN, D), lambda i: (i + OFF, 0)),
            pl.BlockSpec((BN * DEG, D), lambda i: (i + OFF, 0)),
            *_wspecs,
        ],
        out_specs=pl.BlockSpec((BN, 2 * D), lambda i: (i + OFF, 0)),
        out_shape=jax.ShapeDtypeStruct((N_NODES, 2 * D), jnp.float32),
    )(x, neigh, W_x, b_x, W_n, b_n)

    return pl.pallas_call(
        _linear_body,
        grid=(S_SC // BN,),
        in_specs=[
            pl.BlockSpec(memory_space=pl.ANY),
            pl.BlockSpec((BN, D), lambda i: (i, 0)),
            pl.BlockSpec((BN, D), lambda i: (i, 0)),
            *_wspecs,
        ],
        out_specs=pl.BlockSpec((BN, 2 * D), lambda i: (i, 0)),
        out_shape=jax.ShapeDtypeStruct((N_NODES, 2 * D), jnp.float32),
        input_output_aliases={0: 0},
    )(out1, x, agg_sc, W_x, b_x, W_n, b_n)


def kernel(x, neigh, W_x, b_x, W_n, b_n):
    return _run(x, neigh, W_x.reshape(D, D), b_x.reshape(1, D),
                W_n.reshape(D, D), b_n.reshape(1, D))


# speedup vs baseline: 1.5922x; 1.0091x over previous
import functools

import jax
import jax.numpy as jnp
from jax import lax
from jax.experimental import pallas as pl
from jax.experimental.pallas import tpu as pltpu
from jax.experimental.pallas import tpu_sc as plsc

N_NODES = 10000
DEG = 32
D = 128

S_SC = 2000
N_TC = N_NODES - S_SC

NC = 2
NS = 16
NW = NC * NS
LANES = 16
NCHUNK = D // LANES

RB = 8
BLK_ROWS = RB * DEG
NBLK = S_SC // RB
STEPS = 2 * (-(-NBLK // (2 * NW)))

_sc_mesh = plsc.VectorSubcoreMesh(core_axis_name="c", subcore_axis_name="s")


def _sc_mean_body(neigh_hbm, out_hbm, buf, outb, isem0, isem1, osem0, osem1):
    wid = lax.axis_index("s") * NC + lax.axis_index("c")
    isems = (isem0, isem1)
    osems = (osem0, osem1)

    def blk_of(step):
        return jnp.minimum(wid + step * NW, NBLK - 1)

    def in_copy(step, b):
        return pltpu.make_async_copy(
            neigh_hbm.at[pl.ds(blk_of(step) * BLK_ROWS, BLK_ROWS), :],
            buf.at[b], isems[b])

    def out_copy(step, b):
        return pltpu.make_async_copy(
            outb.at[b], out_hbm.at[pl.ds(blk_of(step) * RB, RB), :], osems[b])

    in_copy(0, 0).start()

    def reduce_block(b):
        bufb = buf.at[b]
        outbb = outb.at[b]

        def row_body(r, carry):
            base = r * DEG
            accs = [bufb[base, pl.ds(LANES * j, LANES)] for j in range(NCHUNK)]
            for k in range(1, DEG):
                for j in range(NCHUNK):
                    accs[j] = accs[j] + bufb[base + k, pl.ds(LANES * j, LANES)]
            for j in range(NCHUNK):
                outbb[r, pl.ds(LANES * j, LANES)] = accs[j] * (1.0 / DEG)
            return carry

        lax.fori_loop(0, RB, row_body, 0)

    def outer(g, carry):
        for b in range(2):
            step = 2 * g + b
            in_copy(step + 1, 1 - b).start()
            in_copy(step, b).wait()
            pl.when(g > 0)(lambda: out_copy(step, b).wait())
            reduce_block(b)
            out_copy(step, b).start()
        return carry

    lax.fori_loop(0, STEPS // 2, outer, 0)

    in_copy(STEPS, 0).wait()
    out_copy(STEPS - 2, 0).wait()
    out_copy(STEPS - 1, 1).wait()


_sc_mean = pl.kernel(
    _sc_mean_body,
    mesh=_sc_mesh,
    out_type=jax.ShapeDtypeStruct((S_SC, D), jnp.float32),
    scratch_types=[
        pltpu.VMEM((2, BLK_ROWS, D), jnp.float32),
        pltpu.VMEM((2, RB, D), jnp.float32),
        pltpu.SemaphoreType.DMA,
        pltpu.SemaphoreType.DMA,
        pltpu.SemaphoreType.DMA,
        pltpu.SemaphoreType.DMA,
    ],
)


BN = 400
OFF = S_SC // BN
BNX = 1000
BNT = 1000


def _mm(a_ref, w_ref, b_ref):
    h = lax.dot_general(a_ref[...], w_ref[...], (((1,), (1,)), ((), ())),
                        preferred_element_type=jnp.float32)
    return jnp.maximum(h + b_ref[...], 0.0)


def _fused_body(x_ref, neigh_ref, wx_ref, bx_ref, wn_ref, bn_ref, out_ref):
    nb = neigh_ref[...].reshape(BN, DEG, D)
    agg = jnp.sum(nb, axis=1) * (1.0 / DEG)
    out_ref[:, :D] = _mm(x_ref, wx_ref, bx_ref)
    h_n = lax.dot_general(agg, wn_ref[...], (((1,), (1,)), ((), ())),
                          preferred_element_type=jnp.float32)
    out_ref[:, D:] = jnp.maximum(h_n + bn_ref[...], 0.0)


def _xlin_body(prev_ref, x_ref, wx_ref, bx_ref, out_ref):
    del prev_ref
    out_ref[...] = _mm(x_ref, wx_ref, bx_ref)


def _nlin_body(prev_ref, agg_ref, wn_ref, bn_ref, out_ref):
    del prev_ref
    out_ref[...] = _mm(agg_ref, wn_ref, bn_ref)


@jax.jit
def _run(x, neigh, W_x, b_x, W_n, b_n):
    agg_sc = _sc_mean(neigh)

    wspec = pl.BlockSpec((D, D), lambda i: (0, 0))
    bspec = pl.BlockSpec((1, D), lambda i: (0, 0))

    out1 = pl.pallas_call(
        _fused_body,
        grid=(N_TC // BN,),
        in_specs=[
            pl.BlockSpec((BN, D), lambda i: (i + OFF, 0)),
            pl.BlockSpec((BN * DEG, D), lambda i: (i + OFF, 0)),
            wspec, bspec, wspec, bspec,
        ],
        out_specs=pl.BlockSpec((BN, 2 * D), lambda i: (i + OFF, 0)),
        out_shape=jax.ShapeDtypeStruct((N_NODES, 2 * D), jnp.float32),
    )(x, neigh, W_x, b_x, W_n, b_n)

    out2 = pl.pallas_call(
        _xlin_body,
        grid=(S_SC // BNX,),
        in_specs=[
            pl.BlockSpec(memory_space=pl.ANY),
            pl.BlockSpec((BNX, D), lambda i: (i, 0)),
            wspec, bspec,
        ],
        out_specs=pl.BlockSpec((BNX, D), lambda i: (i, 0)),
        out_shape=jax.ShapeDtypeStruct((N_NODES, 2 * D), jnp.float32),
        input_output_aliases={0: 0},
    )(out1, x, W_x, b_x)

    return pl.pallas_call(
        _nlin_body,
        grid=(S_SC // BNT,),
        in_specs=[
            pl.BlockSpec(memory_space=pl.ANY),
            pl.BlockSpec((BNT, D), lambda i: (i, 0)),
            wspec, bspec,
        ],
        out_specs=pl.BlockSpec((BNT, D), lambda i: (i, 1)),
        out_shape=jax.ShapeDtypeStruct((N_NODES, 2 * D), jnp.float32),
        input_output_aliases={0: 0},
    )(out2, agg_sc, W_n, b_n)


def kernel(x, neigh, W_x, b_x, W_n, b_n):
    return _run(x, neigh, W_x.reshape(D, D), b_x.reshape(1, D),
                W_n.reshape(D, D), b_n.reshape(1, D))


# speedup vs baseline: 2.4253x; 1.5232x over previous
import jax
import jax.numpy as jnp
from jax import lax
from jax.experimental import pallas as pl

N_NODES = 10000
DEG = 32
D = 128
BN = 400


def _fused_body(x_ref, neigh_ref, wx_ref, bx_ref, wn_ref, bn_ref, out_ref):
    nb = neigh_ref[...].reshape(BN, DEG, D)
    agg = jnp.sum(nb, axis=1) * (1.0 / DEG)
    h_x = lax.dot_general(
        x_ref[...], wx_ref[...], (((1,), (1,)), ((), ())),
        preferred_element_type=jnp.float32)
    h_n = lax.dot_general(
        agg, wn_ref[...], (((1,), (1,)), ((), ())),
        preferred_element_type=jnp.float32)
    out_ref[:, :D] = jnp.maximum(h_x + bx_ref[...], 0.0)
    out_ref[:, D:] = jnp.maximum(h_n + bn_ref[...], 0.0)


@jax.jit
def _fused(x, neigh, W_x, b_x, W_n, b_n):
    return pl.pallas_call(
        _fused_body,
        grid=(N_NODES // BN,),
        in_specs=[
            pl.BlockSpec((BN, D), lambda i: (i, 0)),
            pl.BlockSpec((BN * DEG, D), lambda i: (i, 0)),
            pl.BlockSpec((D, D), lambda i: (0, 0)),
            pl.BlockSpec((1, D), lambda i: (0, 0)),
            pl.BlockSpec((D, D), lambda i: (0, 0)),
            pl.BlockSpec((1, D), lambda i: (0, 0)),
        ],
        out_specs=pl.BlockSpec((BN, 2 * D), lambda i: (i, 0)),
        out_shape=jax.ShapeDtypeStruct((N_NODES, 2 * D), jnp.float32),
    )(x, neigh, W_x, b_x, W_n, b_n)


def kernel(x, neigh, W_x, b_x, W_n, b_n):
    return _fused(x, neigh, W_x.reshape(D, D), b_x.reshape(1, D),
                  W_n.reshape(D, D), b_n.reshape(1, D))
